# baseline (speedup 1.0000x reference)
import functools

import jax
import jax.numpy as jnp
from jax import lax
from jax.experimental import pallas as pl
from jax.experimental.pallas import tpu as pltpu
from jax.experimental.pallas import tpu_sc as plsc

_B, _N, _C = 4, 1024, 768
_T = _B * _N
_NW = 32
_RPW = _T // _NW
_WPB = _N // _RPW


def _sc_route_x1(im1, im2, m1, thr):
    mesh = plsc.VectorSubcoreMesh(core_axis_name="c", subcore_axis_name="s")

    @functools.partial(
        pl.kernel,
        out_type=jax.ShapeDtypeStruct((_B, _N, _C), jnp.float32),
        mesh=mesh,
        scratch_types=[
            pltpu.VMEM((_RPW,), jnp.float32),
            pltpu.VMEM((16,), jnp.float32),
            pltpu.SemaphoreType.DMA,
        ],
        compiler_params=pltpu.CompilerParams(use_tc_tiling_on_sc=True),
    )
    def k(im1_hbm, im2_hbm, m1_hbm, thr_hbm, x1_hbm, mv, tv, sem):
        wid = lax.axis_index("s") * 2 + lax.axis_index("c")
        b = wid // _WPB
        n0 = (wid % _WPB) * _RPW
        pltpu.sync_copy(m1_hbm.at[pl.ds(wid * _RPW, _RPW)], mv)
        pltpu.sync_copy(thr_hbm, tv)
        t = tv[...][0]

        def grp(g, carry):
            mvec = mv[pl.ds(g * 16, 16)]
            for j in range(16):
                keep = mvec[j] >= t
                n = n0 + g * 16 + j

                @pl.when(keep)
                def _():
                    pltpu.async_copy(
                        im1_hbm.at[b, n, :], x1_hbm.at[b, n, :], sem)

                @pl.when(jnp.logical_not(keep))
                def _():
                    pltpu.async_copy(
                        im2_hbm.at[b, n, :], x1_hbm.at[b, n, :], sem)

            return carry

        lax.fori_loop(0, _RPW // 16, grp, 0)
        pltpu.make_async_copy(
            im1_hbm.at[b, pl.ds(n0, _RPW), :],
            x1_hbm.at[b, pl.ds(n0, _RPW), :], sem).wait()

    return k(im1, im2, m1, thr)


_BLKN = 512


def _tc_body(thr_ref, m2_ref, a_ref, b_ref, x2_ref):
    t = thr_ref[0]
    k2 = m2_ref[...] >= t
    x2_ref[...] = jnp.where(k2, b_ref[...], a_ref[...])


def _tc_select_x2(im1, im2, m2col, thr):
    grid = (_B, _N // _BLKN)
    return pl.pallas_call(
        _tc_body,
        grid=grid,
        in_specs=[
            pl.BlockSpec(memory_space=pltpu.SMEM),
            pl.BlockSpec((1, _BLKN, 1), lambda i, j: (i, j, 0)),
            pl.BlockSpec((1, _BLKN, _C), lambda i, j: (i, j, 0)),
            pl.BlockSpec((1, _BLKN, _C), lambda i, j: (i, j, 0)),
        ],
        out_specs=pl.BlockSpec((1, _BLKN, _C), lambda i, j: (i, j, 0)),
        out_shape=jax.ShapeDtypeStruct((_B, _N, _C), jnp.float32),
        compiler_params=pltpu.CompilerParams(
            dimension_semantics=("arbitrary", "arbitrary")),
    )(thr, m2col, im1, im2)


def kernel(im1, im2, mask1, mask2, mask_threshold):
    m1f = mask1.reshape(_T)
    m2col = mask2.reshape(_B, _N, 1)
    thr16 = jnp.full((16,), mask_threshold, jnp.float32)
    thr1 = jnp.full((1,), mask_threshold, jnp.float32)
    x1 = _sc_route_x1(im1, im2, m1f, thr16)
    x2 = _tc_select_x2(im1, im2, m2col, thr1)
    return x1, x2


# speedup vs baseline: 4.7445x; 4.7445x over previous
import functools

import jax
import jax.numpy as jnp
from jax import lax
from jax.experimental import pallas as pl
from jax.experimental.pallas import tpu as pltpu
from jax.experimental.pallas import tpu_sc as plsc

_B, _N, _C = 4, 1024, 768
_T = _B * _N
_NW = 32
_RPW = _T // _NW
_WPB = _N // _RPW


_RCH = 16
_NCH = _RPW // _RCH
_CG = _C // 16


def _sc_route_x1(im1f, im2f, m1, thr):
    mesh = plsc.VectorSubcoreMesh(core_axis_name="c", subcore_axis_name="s")

    @functools.partial(
        pl.kernel,
        out_type=jax.ShapeDtypeStruct((_T, _C), jnp.float32),
        mesh=mesh,
        scratch_types=[
            pltpu.VMEM((_RCH, _C), jnp.float32),
            pltpu.VMEM((_RCH, _C), jnp.float32),
            pltpu.VMEM((_RCH, _C), jnp.float32),
            pltpu.VMEM((_RCH,), jnp.int32),
            pltpu.VMEM((_RPW,), jnp.float32),
            pltpu.VMEM((16,), jnp.float32),
            pltpu.SemaphoreType.DMA,
        ],
        compiler_params=pltpu.CompilerParams(use_tc_tiling_on_sc=True),
    )
    def k(im1_hbm, im2_hbm, m1_hbm, thr_hbm, x1_hbm, av, bv, xv, iv, mv, tv,
          sem):
        wid = lax.axis_index("s") * 2 + lax.axis_index("c")
        t0 = wid * _RPW
        pltpu.sync_copy(m1_hbm.at[pl.ds(t0, _RPW)], mv)
        pltpu.sync_copy(thr_hbm, tv)
        tvec = tv[...]

        def chunk(c, carry):
            base = t0 + c * _RCH
            iv[...] = base + lax.iota(jnp.int32, 16)
            pltpu.async_copy(im1_hbm.at[iv], av, sem).wait()
            pltpu.async_copy(im2_hbm.at[iv], bv, sem).wait()
            kv = jnp.where(mv[pl.ds(c * _RCH, _RCH)] >= tvec,
                           jnp.full((16,), -1, jnp.int32),
                           jnp.full((16,), 0, jnp.int32))
            for r in range(_RCH):
                krow = jnp.full((16,), kv[r], jnp.int32)
                nrow = ~krow
                for j in range(_CG):
                    a = lax.bitcast_convert_type(
                        av[r, pl.ds(j * 16, 16)], jnp.int32)
                    b = lax.bitcast_convert_type(
                        bv[r, pl.ds(j * 16, 16)], jnp.int32)
                    xv[r, pl.ds(j * 16, 16)] = lax.bitcast_convert_type(
                        (a & krow) | (b & nrow), jnp.float32)
            pltpu.async_copy(xv, x1_hbm.at[iv], sem).wait()
            return carry

        lax.fori_loop(0, _NCH, chunk, 0)

    return k(im1f, im2f, m1, thr).reshape(_B, _N, _C)


_BLKN = 512


def _tc_body(thr_ref, m2_ref, a_ref, b_ref, x2_ref):
    t = thr_ref[0]
    k2 = m2_ref[...] >= t
    x2_ref[...] = jnp.where(k2, b_ref[...], a_ref[...])


def _tc_select_x2(im1, im2, m2col, thr):
    grid = (_B, _N // _BLKN)
    return pl.pallas_call(
        _tc_body,
        grid=grid,
        in_specs=[
            pl.BlockSpec(memory_space=pltpu.SMEM),
            pl.BlockSpec((1, _BLKN, 1), lambda i, j: (i, j, 0)),
            pl.BlockSpec((1, _BLKN, _C), lambda i, j: (i, j, 0)),
            pl.BlockSpec((1, _BLKN, _C), lambda i, j: (i, j, 0)),
        ],
        out_specs=pl.BlockSpec((1, _BLKN, _C), lambda i, j: (i, j, 0)),
        out_shape=jax.ShapeDtypeStruct((_B, _N, _C), jnp.float32),
        compiler_params=pltpu.CompilerParams(
            dimension_semantics=("arbitrary", "arbitrary")),
    )(thr, m2col, im1, im2)


def kernel(im1, im2, mask1, mask2, mask_threshold):
    m1f = mask1.reshape(_T)
    m2col = mask2.reshape(_B, _N, 1)
    thr16 = jnp.full((16,), mask_threshold, jnp.float32)
    thr1 = jnp.full((1,), mask_threshold, jnp.float32)
    x1 = _sc_route_x1(im1.reshape(_T, _C), im2.reshape(_T, _C), m1f, thr16)
    x2 = _tc_select_x2(im1, im2, m2col, thr1)
    return x1, x2


# speedup vs baseline: 7.9304x; 1.6715x over previous
import functools

import jax
import jax.numpy as jnp
from jax import lax
from jax.experimental import pallas as pl
from jax.experimental.pallas import tpu as pltpu
from jax.experimental.pallas import tpu_sc as plsc

_B, _N, _C = 4, 1024, 768
_T = _B * _N
_NW = 32
_RPW = _T // _NW
_WPB = _N // _RPW


_RCH = 16
_NCH = _RPW // _RCH
_CG = _C // 16


def _sc_route_x1(im1f, im2f, m1, thr):
    mesh = plsc.VectorSubcoreMesh(core_axis_name="c", subcore_axis_name="s")

    _NSET = 3

    @functools.partial(
        pl.kernel,
        out_type=jax.ShapeDtypeStruct((_T, _C), jnp.float32),
        mesh=mesh,
        scratch_types=(
            [pltpu.VMEM((_RCH, _C), jnp.float32)] * _NSET
            + [pltpu.VMEM((_RCH, _C), jnp.float32)] * _NSET
            + [pltpu.VMEM((_RCH, _C), jnp.float32)] * _NSET
            + [pltpu.VMEM((_RCH,), jnp.int32)] * _NSET
            + [pltpu.VMEM((_RPW,), jnp.float32),
               pltpu.VMEM((16,), jnp.float32)]
            + [pltpu.SemaphoreType.DMA] * (2 * _NSET)
        ),
        compiler_params=pltpu.CompilerParams(use_tc_tiling_on_sc=True),
    )
    def k(im1_hbm, im2_hbm, m1_hbm, thr_hbm, x1_hbm, *scr):
        av = scr[0:_NSET]
        bv = scr[_NSET:2 * _NSET]
        xv = scr[2 * _NSET:3 * _NSET]
        iv = scr[3 * _NSET:4 * _NSET]
        mv, tv = scr[4 * _NSET], scr[4 * _NSET + 1]
        sg = scr[4 * _NSET + 2:4 * _NSET + 2 + _NSET]
        ss = scr[4 * _NSET + 2 + _NSET:]

        wid = lax.axis_index("s") * 2 + lax.axis_index("c")
        t0 = wid * _RPW
        pltpu.sync_copy(m1_hbm.at[pl.ds(t0, _RPW)], mv)
        pltpu.sync_copy(thr_hbm, tv)
        tvec = tv[...]
        dnums = lax.GatherDimensionNumbers(
            offset_dims=(), collapsed_slice_dims=(0,), start_index_map=(0,))

        hg, hs = {}, {}

        def issue(c):
            s = c % _NSET
            iv[s][...] = t0 + c * _RCH + lax.iota(jnp.int32, 16)
            hg[c] = (pltpu.async_copy(im1_hbm.at[iv[s]], av[s], sg[s]),
                     pltpu.async_copy(im2_hbm.at[iv[s]], bv[s], sg[s]))

        def compute(c):
            s = c % _NSET
            kv = jnp.where(mv[pl.ds(c * _RCH, _RCH)] >= tvec,
                           jnp.full((16,), -1, jnp.int32),
                           jnp.full((16,), 0, jnp.int32))

            def row(r, carry2):
                krow = lax.gather(
                    kv, jnp.full((16, 1), r, jnp.int32), dnums, (1,),
                    mode=lax.GatherScatterMode.PROMISE_IN_BOUNDS)
                nrow = ~krow
                for j in range(_CG):
                    a = lax.bitcast_convert_type(
                        av[s][r, pl.ds(j * 16, 16)], jnp.int32)
                    b = lax.bitcast_convert_type(
                        bv[s][r, pl.ds(j * 16, 16)], jnp.int32)
                    xv[s][r, pl.ds(j * 16, 16)] = lax.bitcast_convert_type(
                        (a & krow) | (b & nrow), jnp.float32)
                return carry2

            lax.fori_loop(0, _RCH, row, 0)

        issue(0)
        issue(1)
        for c in range(_NCH):
            s = c % _NSET
            h1, h2 = hg.pop(c)
            h1.wait()
            h2.wait()
            compute(c)
            hs[c] = pltpu.async_copy(xv[s], x1_hbm.at[iv[s]], ss[s])
            if c + 2 < _NCH:
                if c - 1 >= 0:
                    hs.pop(c - 1).wait()
                issue(c + 2)
        for c in (_NCH - 2, _NCH - 1):
            hs.pop(c).wait()

    return k(im1f, im2f, m1, thr).reshape(_B, _N, _C)


_BLKN = 512


def _tc_body(thr_ref, m2_ref, a_ref, b_ref, x2_ref):
    t = thr_ref[0]
    k2 = m2_ref[...] >= t
    x2_ref[...] = jnp.where(k2, b_ref[...], a_ref[...])


def _tc_select_x2(im1, im2, m2col, thr):
    grid = (_B, _N // _BLKN)
    return pl.pallas_call(
        _tc_body,
        grid=grid,
        in_specs=[
            pl.BlockSpec(memory_space=pltpu.SMEM),
            pl.BlockSpec((1, _BLKN, 1), lambda i, j: (i, j, 0)),
            pl.BlockSpec((1, _BLKN, _C), lambda i, j: (i, j, 0)),
            pl.BlockSpec((1, _BLKN, _C), lambda i, j: (i, j, 0)),
        ],
        out_specs=pl.BlockSpec((1, _BLKN, _C), lambda i, j: (i, j, 0)),
        out_shape=jax.ShapeDtypeStruct((_B, _N, _C), jnp.float32),
        compiler_params=pltpu.CompilerParams(
            dimension_semantics=("arbitrary", "arbitrary")),
    )(thr, m2col, im1, im2)


def kernel(im1, im2, mask1, mask2, mask_threshold):
    m1f = mask1.reshape(_T)
    m2col = mask2.reshape(_B, _N, 1)
    thr16 = jnp.full((16,), mask_threshold, jnp.float32)
    thr1 = jnp.full((1,), mask_threshold, jnp.float32)
    x1 = _sc_route_x1(im1.reshape(_T, _C), im2.reshape(_T, _C), m1f, thr16)
    x2 = _tc_select_x2(im1, im2, m2col, thr1)
    return x1, x2


# speedup vs baseline: 8.0250x; 1.0119x over previous
import functools

import jax
import jax.numpy as jnp
from jax import lax
from jax.experimental import pallas as pl
from jax.experimental.pallas import tpu as pltpu
from jax.experimental.pallas import tpu_sc as plsc

_B, _N, _C = 4, 1024, 768
_T = _B * _N
_NW = 32
_RPW = _T // _NW
_WPB = _N // _RPW


_RCH = 16
_NCH = _RPW // _RCH
_CG = _C // 16


def _sc_route_x1(im1f, im2f, m1, thr):
    mesh = plsc.VectorSubcoreMesh(core_axis_name="c", subcore_axis_name="s")

    _NSET = 3

    @functools.partial(
        pl.kernel,
        out_type=jax.ShapeDtypeStruct((_T, _C), jnp.float32),
        mesh=mesh,
        scratch_types=(
            [pltpu.VMEM((_RCH, _C), jnp.float32)] * _NSET
            + [pltpu.VMEM((_RCH, _C), jnp.float32)] * _NSET
            + [pltpu.VMEM((_RCH, _C), jnp.float32)] * _NSET
            + [pltpu.VMEM((_RPW,), jnp.float32),
               pltpu.VMEM((16,), jnp.float32)]
            + [pltpu.SemaphoreType.DMA] * (2 * _NSET)
        ),
        compiler_params=pltpu.CompilerParams(use_tc_tiling_on_sc=True),
    )
    def k(im1_hbm, im2_hbm, m1_hbm, thr_hbm, x1_hbm, *scr):
        av = scr[0:_NSET]
        bv = scr[_NSET:2 * _NSET]
        xv = scr[2 * _NSET:3 * _NSET]
        mv, tv = scr[3 * _NSET], scr[3 * _NSET + 1]
        sg = scr[3 * _NSET + 2:3 * _NSET + 2 + _NSET]
        ss = scr[3 * _NSET + 2 + _NSET:]

        wid = lax.axis_index("s") * 2 + lax.axis_index("c")
        t0 = wid * _RPW
        pltpu.sync_copy(m1_hbm.at[pl.ds(t0, _RPW)], mv)
        pltpu.sync_copy(thr_hbm, tv)
        tvec = tv[...]
        dnums = lax.GatherDimensionNumbers(
            offset_dims=(), collapsed_slice_dims=(0,), start_index_map=(0,))

        hg, hs = {}, {}

        def issue(c):
            s = c % _NSET
            sl = pl.ds(t0 + c * _RCH, _RCH)
            hg[c] = (pltpu.async_copy(im1_hbm.at[sl, :], av[s], sg[s]),
                     pltpu.async_copy(im2_hbm.at[sl, :], bv[s], sg[s]))

        def compute(c):
            s = c % _NSET
            kv = jnp.where(mv[pl.ds(c * _RCH, _RCH)] >= tvec,
                           jnp.full((16,), -1, jnp.int32),
                           jnp.full((16,), 0, jnp.int32))

            def row(r, carry2):
                krow = lax.gather(
                    kv, jnp.full((16, 1), r, jnp.int32), dnums, (1,),
                    mode=lax.GatherScatterMode.PROMISE_IN_BOUNDS)
                nrow = ~krow
                for j in range(_CG):
                    a = lax.bitcast_convert_type(
                        av[s][r, pl.ds(j * 16, 16)], jnp.int32)
                    b = lax.bitcast_convert_type(
                        bv[s][r, pl.ds(j * 16, 16)], jnp.int32)
                    xv[s][r, pl.ds(j * 16, 16)] = lax.bitcast_convert_type(
                        (a & krow) | (b & nrow), jnp.float32)
                return carry2

            lax.fori_loop(0, _RCH, row, 0)

        issue(0)
        issue(1)
        for c in range(_NCH):
            s = c % _NSET
            h1, h2 = hg.pop(c)
            h1.wait()
            h2.wait()
            compute(c)
            hs[c] = pltpu.async_copy(
                xv[s], x1_hbm.at[pl.ds(t0 + c * _RCH, _RCH), :], ss[s])
            if c + 2 < _NCH:
                if c - 1 >= 0:
                    hs.pop(c - 1).wait()
                issue(c + 2)
        for c in (_NCH - 2, _NCH - 1):
            hs.pop(c).wait()

    return k(im1f, im2f, m1, thr).reshape(_B, _N, _C)


_BLKN = 512


def _tc_body(thr_ref, m2_ref, a_ref, b_ref, x2_ref):
    t = thr_ref[0]
    k2 = m2_ref[...] >= t
    x2_ref[...] = jnp.where(k2, b_ref[...], a_ref[...])


def _tc_select_x2(im1, im2, m2col, thr):
    grid = (_B, _N // _BLKN)
    return pl.pallas_call(
        _tc_body,
        grid=grid,
        in_specs=[
            pl.BlockSpec(memory_space=pltpu.SMEM),
            pl.BlockSpec((1, _BLKN, 1), lambda i, j: (i, j, 0)),
            pl.BlockSpec((1, _BLKN, _C), lambda i, j: (i, j, 0)),
            pl.BlockSpec((1, _BLKN, _C), lambda i, j: (i, j, 0)),
        ],
        out_specs=pl.BlockSpec((1, _BLKN, _C), lambda i, j: (i, j, 0)),
        out_shape=jax.ShapeDtypeStruct((_B, _N, _C), jnp.float32),
        compiler_params=pltpu.CompilerParams(
            dimension_semantics=("arbitrary", "arbitrary")),
    )(thr, m2col, im1, im2)


def kernel(im1, im2, mask1, mask2, mask_threshold):
    m1f = mask1.reshape(_T)
    m2col = mask2.reshape(_B, _N, 1)
    thr16 = jnp.full((16,), mask_threshold, jnp.float32)
    thr1 = jnp.full((1,), mask_threshold, jnp.float32)
    x1 = _sc_route_x1(im1.reshape(_T, _C), im2.reshape(_T, _C), m1f, thr16)
    x2 = _tc_select_x2(im1, im2, m2col, thr1)
    return x1, x2
